# baseline (speedup 1.0000x reference)
import jax
import jax.numpy as jnp
from jax.experimental import pallas as pl
from jax.experimental.pallas import tpu as pltpu

_BR = 200
_NB = 4


def _gconv_body(x_ref, w_ref, wl_ref, b_ref, a_hbm, o_hbm,
                s_ref, abufs, obufs, in_sems, out_sems):
    n = a_hbm.shape[0]
    npieces = n // _BR

    def in_copy(p, slot):
        return pltpu.make_async_copy(
            a_hbm.at[pl.ds(p * _BR, _BR), :], abufs.at[slot], in_sems.at[slot])

    def out_copy(p, slot):
        return pltpu.make_async_copy(
            obufs.at[slot], o_hbm.at[pl.ds(p * _BR, _BR), :], out_sems.at[slot])

    for b in range(_NB):
        in_copy(b, b).start()

    s_ref[...] = jnp.dot(
        x_ref[...].astype(jnp.bfloat16), w_ref[...],
        preferred_element_type=jnp.float32,
    ).astype(jnp.bfloat16)

    def step(p, _):
        islot = jax.lax.rem(p, _NB)
        oslot = jax.lax.rem(p, 2)
        in_copy(p, islot).wait()
        acc = jnp.dot(
            abufs[islot].astype(jnp.bfloat16), s_ref[...],
            preferred_element_type=jnp.float32,
        )
        x_blk = x_ref[pl.ds(p * _BR, _BR), :].astype(jnp.bfloat16)
        loop = jnp.dot(x_blk, wl_ref[...], preferred_element_type=jnp.float32)

        @pl.when(p >= 2)
        def _drain_out():
            out_copy(p - 2, oslot).wait()

        obufs[oslot] = acc + loop + b_ref[...]
        out_copy(p, oslot).start()

        @pl.when(p + _NB < npieces)
        def _refill():
            in_copy(p + _NB, islot).start()

        return 0

    jax.lax.fori_loop(0, npieces, step, 0)
    out_copy(npieces - 2, jax.lax.rem(npieces - 2, 2)).wait()
    out_copy(npieces - 1, jax.lax.rem(npieces - 1, 2)).wait()


def kernel(inputs, adj_mat, weight, loop_weight, bias):
    n, d_in = inputs.shape
    d_out = weight.shape[1]

    w16 = weight.astype(jnp.bfloat16)
    wl16 = loop_weight.astype(jnp.bfloat16)
    b2 = bias.reshape(1, d_out)

    return pl.pallas_call(
        _gconv_body,
        in_specs=[
            pl.BlockSpec((n, d_in), lambda: (0, 0)),
            pl.BlockSpec((d_in, d_out), lambda: (0, 0)),
            pl.BlockSpec((d_in, d_out), lambda: (0, 0)),
            pl.BlockSpec((1, d_out), lambda: (0, 0)),
            pl.BlockSpec(memory_space=pl.ANY),
        ],
        out_specs=pl.BlockSpec(memory_space=pl.ANY),
        out_shape=jax.ShapeDtypeStruct((n, d_out), jnp.float32),
        compiler_params=pltpu.CompilerParams(vmem_limit_bytes=110 * 1024 * 1024),
        scratch_shapes=[
            pltpu.VMEM((n, d_out), jnp.bfloat16),
            pltpu.VMEM((_NB, _BR, n), jnp.float32),
            pltpu.VMEM((2, _BR, d_out), jnp.float32),
            pltpu.SemaphoreType.DMA((_NB,)),
            pltpu.SemaphoreType.DMA((2,)),
        ],
    )(inputs, w16, wl16, b2, adj_mat)


# speedup vs baseline: 1.0228x; 1.0228x over previous
import jax
import jax.numpy as jnp
from jax.experimental import pallas as pl
from jax.experimental.pallas import tpu as pltpu

_BM = 400


def _gconv_body(a_ref, x_ref, w_ref, wl_ref, b_ref, o_ref, s_ref):
    i = pl.program_id(0)

    @pl.when(i == 0)
    def _init_support():
        s_ref[...] = jnp.dot(
            x_ref[...].astype(jnp.bfloat16), w_ref[...],
            preferred_element_type=jnp.float32,
        ).astype(jnp.bfloat16)

    acc = jnp.dot(
        a_ref[...].astype(jnp.bfloat16), s_ref[...],
        preferred_element_type=jnp.float32,
    )
    x_blk = x_ref[pl.ds(i * _BM, _BM), :].astype(jnp.bfloat16)
    loop = jnp.dot(x_blk, wl_ref[...], preferred_element_type=jnp.float32)
    o_ref[...] = acc + loop + b_ref[...]


def kernel(inputs, adj_mat, weight, loop_weight, bias):
    n, d_in = inputs.shape
    d_out = weight.shape[1]

    w16 = weight.astype(jnp.bfloat16)
    wl16 = loop_weight.astype(jnp.bfloat16)
    b2 = bias.reshape(1, d_out)

    return pl.pallas_call(
        _gconv_body,
        grid=(n // _BM,),
        in_specs=[
            pl.BlockSpec((_BM, n), lambda i: (i, 0)),
            pl.BlockSpec((n, d_in), lambda i: (0, 0)),
            pl.BlockSpec((d_in, d_out), lambda i: (0, 0)),
            pl.BlockSpec((d_in, d_out), lambda i: (0, 0)),
            pl.BlockSpec((1, d_out), lambda i: (0, 0)),
        ],
        out_specs=pl.BlockSpec((_BM, d_out), lambda i: (i, 0)),
        out_shape=jax.ShapeDtypeStruct((n, d_out), jnp.float32),
        compiler_params=pltpu.CompilerParams(vmem_limit_bytes=110 * 1024 * 1024),
        scratch_shapes=[pltpu.VMEM((n, d_out), jnp.bfloat16)],
    )(adj_mat, inputs, w16, wl16, b2)
